# Initial kernel scaffold; baseline (speedup 1.0000x reference)
import jax
import jax.numpy as jnp
from jax.experimental import pallas as pl


def kernel(x, group_idx, mean, std):
    raise NotImplementedError("write your pallas kernel here")


# speedup vs baseline: 5.1946x; 5.1946x over previous
import functools

import jax
import jax.numpy as jnp
from jax import lax
from jax.experimental import pallas as pl

_BATCH = 16384
_FEAT = 1024
_GROUPS = 64
_BR = 1024


def _body(idx_ref, x_ref, mean_ref, std_ref, out_ref):
    g = idx_ref[0, 0, :]
    oh = (g[:, None] == lax.broadcasted_iota(jnp.int32, (_BR, _GROUPS), 1))
    oh = oh.astype(jnp.float32)
    rtab = 1.0 / (std_ref[...] + 1e-5)
    m = jnp.dot(oh, mean_ref[...], preferred_element_type=jnp.float32)
    r = jnp.dot(oh, rtab, preferred_element_type=jnp.float32)
    out_ref[...] = (x_ref[...] - m) * r


@functools.partial(jax.jit, static_argnames=())
def kernel(x, group_idx, mean, std):
    grid = _BATCH // _BR
    idx3 = group_idx.astype(jnp.int32).reshape(grid, 1, _BR)
    return pl.pallas_call(
        _body,
        grid=(grid,),
        in_specs=[
            pl.BlockSpec((1, 1, _BR), lambda i: (i, 0, 0)),
            pl.BlockSpec((_BR, _FEAT), lambda i: (i, 0)),
            pl.BlockSpec((_GROUPS, _FEAT), lambda i: (0, 0)),
            pl.BlockSpec((_GROUPS, _FEAT), lambda i: (0, 0)),
        ],
        out_specs=pl.BlockSpec((_BR, _FEAT), lambda i: (i, 0)),
        out_shape=jax.ShapeDtypeStruct((_BATCH, _FEAT), jnp.float32),
    )(idx3, x, mean, std)
